# baseline (speedup 1.0000x reference)
import functools

import jax
import jax.numpy as jnp
from jax import lax
from jax.experimental import pallas as pl
from jax.experimental.pallas import tpu as pltpu
from jax.experimental.pallas import tpu_sc as plsc

NUM_DATASETS = 100
EMBED_DIM = 128
BATCH = 16384

_info = plsc.get_sparse_core_info()
_NC, _NS = _info.num_cores, _info.num_subcores
_NW = _NC * _NS
_B_PER_W = BATCH // _NW

_CH = 128
_NCH = _B_PER_W // _CH


def _build():
  mesh = plsc.VectorSubcoreMesh(core_axis_name="c", subcore_axis_name="s")

  scratch = (
      [
          pltpu.VMEM((_B_PER_W,), jnp.int32),
          pltpu.VMEM_SHARED((NUM_DATASETS, EMBED_DIM), jnp.float32),
      ]
      + [pltpu.VMEM((_CH, EMBED_DIM), jnp.float32) for _ in range(_NCH)]
      + [pltpu.SemaphoreType.DMA for _ in range(_NCH)]
      + [pltpu.SemaphoreType.DMA, pltpu.SemaphoreType.DMA]
  )

  @functools.partial(
      pl.kernel,
      mesh=mesh,
      out_type=jax.ShapeDtypeStruct((BATCH, EMBED_DIM), jnp.float32),
      scratch_types=scratch,
  )
  def gather_kernel(idx_hbm, table_hbm, out_hbm, *refs):
    idx_v = refs[0]
    table_sh = refs[1]
    bufs = refs[2 : 2 + _NCH]
    gsems = refs[2 + _NCH : 2 + 2 * _NCH]
    osem = refs[2 + 2 * _NCH]
    isem = refs[3 + 2 * _NCH]

    sid = lax.axis_index("s")
    wid = sid * _NC + lax.axis_index("c")
    base = wid * _B_PER_W

    @pl.when(sid == 0)
    def _stage_table():
      pltpu.sync_copy(table_hbm, table_sh)

    idx_copy = pltpu.async_copy(idx_hbm.at[pl.ds(base, _B_PER_W)], idx_v, isem)
    plsc.subcore_barrier()
    idx_copy.wait()

    gathers = [
        pltpu.async_copy(table_sh.at[idx_v.at[pl.ds(c * _CH, _CH)]], bufs[c], gsems[c])
        for c in range(_NCH)
    ]
    outs = []
    for c in range(_NCH):
      gathers[c].wait()
      outs.append(
          pltpu.async_copy(bufs[c], out_hbm.at[pl.ds(base + c * _CH, _CH)], osem)
      )
    for o in outs:
      o.wait()

  return gather_kernel


_gather = jax.jit(_build())


def kernel(dataset_indices, embedding_table):
  return _gather(dataset_indices, embedding_table)


# speedup vs baseline: 1.0003x; 1.0003x over previous
import functools

import jax
import jax.numpy as jnp
from jax import lax
from jax.experimental import pallas as pl
from jax.experimental.pallas import tpu as pltpu
from jax.experimental.pallas import tpu_sc as plsc

NUM_DATASETS = 100
EMBED_DIM = 128
BATCH = 16384

_info = plsc.get_sparse_core_info()
_NC, _NS = _info.num_cores, _info.num_subcores
_NW = _NC * _NS
_B_PER_W = BATCH // _NW

_CHUNKS = (32, 96, 128, 128, 128)
assert sum(_CHUNKS) == _B_PER_W
_NCH = len(_CHUNKS)
_OFFS = tuple(sum(_CHUNKS[:i]) for i in range(_NCH))


def _build():
  mesh = plsc.VectorSubcoreMesh(core_axis_name="c", subcore_axis_name="s")

  scratch = (
      [
          pltpu.VMEM((_B_PER_W,), jnp.int32),
          pltpu.VMEM_SHARED((NUM_DATASETS, EMBED_DIM), jnp.float32),
      ]
      + [pltpu.VMEM((cs, EMBED_DIM), jnp.float32) for cs in _CHUNKS]
      + [pltpu.SemaphoreType.DMA for _ in range(_NCH)]
      + [pltpu.SemaphoreType.DMA, pltpu.SemaphoreType.DMA]
  )

  @functools.partial(
      pl.kernel,
      mesh=mesh,
      out_type=jax.ShapeDtypeStruct((BATCH, EMBED_DIM), jnp.float32),
      scratch_types=scratch,
  )
  def gather_kernel(idx_hbm, table_hbm, out_hbm, *refs):
    idx_v = refs[0]
    table_sh = refs[1]
    bufs = refs[2 : 2 + _NCH]
    gsems = refs[2 + _NCH : 2 + 2 * _NCH]
    osem = refs[2 + 2 * _NCH]
    isem = refs[3 + 2 * _NCH]

    sid = lax.axis_index("s")
    wid = sid * _NC + lax.axis_index("c")
    base = wid * _B_PER_W

    @pl.when(sid == 0)
    def _stage_table():
      pltpu.sync_copy(table_hbm, table_sh)

    idx_copy = pltpu.async_copy(idx_hbm.at[pl.ds(base, _B_PER_W)], idx_v, isem)
    plsc.subcore_barrier()
    idx_copy.wait()

    gathers = [
        pltpu.async_copy(
            table_sh.at[idx_v.at[pl.ds(_OFFS[c], _CHUNKS[c])]], bufs[c], gsems[c]
        )
        for c in range(_NCH)
    ]
    outs = []
    for c in range(_NCH):
      gathers[c].wait()
      outs.append(
          pltpu.async_copy(
              bufs[c], out_hbm.at[pl.ds(base + _OFFS[c], _CHUNKS[c])], osem
          )
      )
    for o in outs:
      o.wait()

  return gather_kernel


_gather = jax.jit(_build())


def kernel(dataset_indices, embedding_table):
  return _gather(dataset_indices, embedding_table)


# speedup vs baseline: 1.0050x; 1.0047x over previous
import functools

import jax
import jax.numpy as jnp
from jax import lax
from jax.experimental import pallas as pl
from jax.experimental.pallas import tpu as pltpu
from jax.experimental.pallas import tpu_sc as plsc

NUM_DATASETS = 100
EMBED_DIM = 128
BATCH = 16384

_info = plsc.get_sparse_core_info()
_NC, _NS = _info.num_cores, _info.num_subcores
_NW = _NC * _NS
_B_PER_W = BATCH // _NW

_CHUNKS = (32, 96, 128, 128, 128)
assert sum(_CHUNKS) == _B_PER_W
_NCH = len(_CHUNKS)
_OFFS = tuple(sum(_CHUNKS[:i]) for i in range(_NCH))


def _build():
  mesh = plsc.VectorSubcoreMesh(core_axis_name="c", subcore_axis_name="s")

  scratch = (
      [
          pltpu.VMEM((_B_PER_W,), jnp.int32),
          pltpu.VMEM_SHARED((NUM_DATASETS, EMBED_DIM), jnp.float32),
      ]
      + [pltpu.VMEM((cs, EMBED_DIM), jnp.float32) for cs in _CHUNKS]
      + [pltpu.SemaphoreType.DMA for _ in range(_NCH)]
      + [pltpu.SemaphoreType.DMA, pltpu.SemaphoreType.DMA,
         pltpu.SemaphoreType.DMA]
  )

  @functools.partial(
      pl.kernel,
      mesh=mesh,
      out_type=jax.ShapeDtypeStruct((BATCH, EMBED_DIM), jnp.float32),
      scratch_types=scratch,
  )
  def gather_kernel(idx_hbm, table_hbm, out_hbm, *refs):
    idx_v = refs[0]
    table_sh = refs[1]
    bufs = refs[2 : 2 + _NCH]
    gsems = refs[2 + _NCH : 2 + 2 * _NCH]
    osems = refs[2 + 2 * _NCH : 4 + 2 * _NCH]
    isem = refs[4 + 2 * _NCH]

    sid = lax.axis_index("s")
    wid = sid * _NC + lax.axis_index("c")
    base = wid * _B_PER_W

    @pl.when(sid == 0)
    def _stage_table():
      pltpu.sync_copy(table_hbm, table_sh)

    idx_copy = pltpu.async_copy(idx_hbm.at[pl.ds(base, _B_PER_W)], idx_v, isem)
    plsc.subcore_barrier()
    idx_copy.wait()

    gathers = [
        pltpu.async_copy(
            table_sh.at[idx_v.at[pl.ds(_OFFS[c], _CHUNKS[c])]], bufs[c], gsems[c]
        )
        for c in range(_NCH)
    ]
    outs = []
    for c in range(_NCH):
      gathers[c].wait()
      outs.append(
          pltpu.async_copy(
              bufs[c], out_hbm.at[pl.ds(base + _OFFS[c], _CHUNKS[c])], osems[c % 2]
          )
      )
    for o in outs:
      o.wait()

  return gather_kernel


_gather = jax.jit(_build())


def kernel(dataset_indices, embedding_table):
  return _gather(dataset_indices, embedding_table)
